# baseline (speedup 1.0000x reference)
import functools

import jax
import jax.numpy as jnp
from jax import lax
from jax.experimental import pallas as pl
from jax.experimental.pallas import tpu as pltpu
from jax.experimental.pallas import tpu_sc as plsc

_EPS = 1e-05
_K = 20
_NSUB = 16


def _sc_segsum(vals, src, dst, n_out):
    N, D = vals.shape
    E = src.shape[0]
    NW = 2 * _NSUB
    epw = E // NW
    nfull = epw // 128
    tail = epw - nfull * 128
    rpt = ((n_out + _NSUB * 8 - 1) // (_NSUB * 8)) * 8
    n_pad = rpt * _NSUB
    zeros = jnp.zeros((rpt, D), jnp.float32)

    scratch = [
        pltpu.VMEM((2, 128), jnp.int32),
        pltpu.VMEM((2, 128), jnp.int32),
        pltpu.VMEM((2, 128, D), jnp.float32),
        pltpu.VMEM_SHARED((n_pad, D), jnp.float32),
        pltpu.SemaphoreType.DMA,
        pltpu.SemaphoreType.DMA,
    ]
    if tail:
        scratch += [
            pltpu.VMEM((tail,), jnp.int32),
            pltpu.VMEM((tail,), jnp.int32),
            pltpu.VMEM((tail, D), jnp.float32),
        ]

    @functools.partial(
        pl.kernel,
        out_type=jax.ShapeDtypeStruct((2, n_pad, D), jnp.float32),
        mesh=plsc.VectorSubcoreMesh(core_axis_name="c", subcore_axis_name="s"),
        scratch_types=scratch,
        compiler_params=pltpu.CompilerParams(use_tc_tiling_on_sc=False),
    )
    def k(vals_hbm, src_hbm, dst_hbm, zeros_hbm, out_hbm,
          src_v, dst_v, rows_v, acc, sem0, sem1, *tailrefs):
        c = lax.axis_index("c")
        s = lax.axis_index("s")
        wid = s * 2 + c
        pltpu.sync_copy(zeros_hbm, acc.at[pl.ds(s * rpt, rpt)])
        plsc.subcore_barrier()
        base = wid * epw

        def fetch(chunk, buf, sem):
            off = base + chunk * 128
            pltpu.sync_copy(src_hbm.at[pl.ds(off, 128)], src_v.at[buf])
            pltpu.sync_copy(dst_hbm.at[pl.ds(off, 128)], dst_v.at[buf])
            pltpu.async_copy(vals_hbm.at[src_v.at[buf]], rows_v.at[buf], sem)

        def drain(buf, sem):
            pltpu.make_async_copy(
                vals_hbm.at[src_v.at[buf]], rows_v.at[buf], sem).wait()
            pltpu.sync_copy(rows_v.at[buf], acc.at[dst_v.at[buf]], add=True)

        fetch(0, 0, sem0)

        def body(j, carry):
            a = 2 * j

            @pl.when(a + 1 < nfull)
            def _():
                fetch(a + 1, 1, sem1)

            drain(0, sem0)

            @pl.when(a + 2 < nfull)
            def _():
                fetch(a + 2, 0, sem0)

            @pl.when(a + 1 < nfull)
            def _():
                drain(1, sem1)

            return carry

        lax.fori_loop(0, (nfull + 1) // 2, body, 0)
        if tail:
            src_t, dst_t, rows_t = tailrefs
            off = base + nfull * 128
            pltpu.sync_copy(src_hbm.at[pl.ds(off, tail)], src_t)
            pltpu.sync_copy(dst_hbm.at[pl.ds(off, tail)], dst_t)
            pltpu.async_copy(vals_hbm.at[src_t], rows_t, sem0).wait()
            pltpu.sync_copy(rows_t, acc.at[dst_t], add=True)
        plsc.subcore_barrier()
        pltpu.sync_copy(acc.at[pl.ds(s * rpt, rpt)],
                        out_hbm.at[c, pl.ds(s * rpt, rpt)])

    return k(vals, src, dst, zeros)


def _sc_gather(vals, idx):
    N, D = vals.shape
    M = idx.shape[0]
    NW = 2 * _NSUB
    nch = M // 128
    tail = M - nch * 128
    cpw = (nch + NW - 1) // NW

    scratch = [
        pltpu.VMEM((2, 128), jnp.int32),
        pltpu.VMEM((2, 128, D), jnp.float32),
        pltpu.SemaphoreType.DMA,
        pltpu.SemaphoreType.DMA,
    ]
    if tail:
        scratch += [
            pltpu.VMEM((tail,), jnp.int32),
            pltpu.VMEM((tail, D), jnp.float32),
        ]

    @functools.partial(
        pl.kernel,
        out_type=jax.ShapeDtypeStruct((M, D), jnp.float32),
        mesh=plsc.VectorSubcoreMesh(core_axis_name="c", subcore_axis_name="s"),
        scratch_types=scratch,
        compiler_params=pltpu.CompilerParams(use_tc_tiling_on_sc=False),
    )
    def k(vals_hbm, idx_hbm, out_hbm, idx_v, rows_v, sem0, sem1, *tailrefs):
        c = lax.axis_index("c")
        s = lax.axis_index("s")
        wid = s * 2 + c

        def fetch(g, buf, sem):
            pltpu.sync_copy(idx_hbm.at[pl.ds(g * 128, 128)], idx_v.at[buf])
            pltpu.async_copy(vals_hbm.at[idx_v.at[buf]], rows_v.at[buf], sem)

        def drain(g, buf, sem):
            pltpu.make_async_copy(
                vals_hbm.at[idx_v.at[buf]], rows_v.at[buf], sem).wait()
            pltpu.sync_copy(rows_v.at[buf], out_hbm.at[pl.ds(g * 128, 128)])

        @pl.when(wid < nch)
        def _():
            fetch(wid, 0, sem0)

        def body(j, carry):
            g0 = wid + 2 * j * NW
            g1 = g0 + NW
            g2 = g1 + NW

            @pl.when(g1 < nch)
            def _():
                fetch(g1, 1, sem1)

            @pl.when(g0 < nch)
            def _():
                drain(g0, 0, sem0)

            @pl.when(g2 < nch)
            def _():
                fetch(g2, 0, sem0)

            @pl.when(g1 < nch)
            def _():
                drain(g1, 1, sem1)

            return carry

        lax.fori_loop(0, (cpw + 1) // 2, body, 0)
        if tail:
            idx_t, rows_t = tailrefs

            @pl.when(wid == NW - 1)
            def _():
                off = nch * 128
                pltpu.sync_copy(idx_hbm.at[pl.ds(off, tail)], idx_t)
                pltpu.async_copy(vals_hbm.at[idx_t], rows_t, sem0).wait()
                pltpu.sync_copy(rows_t, out_hbm.at[pl.ds(off, tail)])

    return k(vals, idx)


def _mlp_layer_kernel(hin_ref, w_ref, b_ref, hout_ref, psum_ref, psq_ref):
    h = jnp.dot(hin_ref[...], w_ref[...], preferred_element_type=jnp.float32) + b_ref[...]
    hout_ref[...] = h
    psum_ref[0, 0, :] = jnp.sum(h, axis=0)
    psq_ref[0, 0, :] = jnp.sum(h * h, axis=0)


def _mlp_layer(hin, W, b, EB=4000):
    M, _ = hin.shape
    Dh = W.shape[1]
    Din = hin.shape[1]
    nblk = M // EB
    h, psum, psq = pl.pallas_call(
        _mlp_layer_kernel,
        grid=(nblk,),
        in_specs=[
            pl.BlockSpec((EB, Din), lambda i: (i, 0)),
            pl.BlockSpec((Din, Dh), lambda i: (0, 0)),
            pl.BlockSpec((1, Dh), lambda i: (0, 0)),
        ],
        out_specs=[
            pl.BlockSpec((EB, Dh), lambda i: (i, 0)),
            pl.BlockSpec((1, 1, Dh), lambda i: (i, 0, 0)),
            pl.BlockSpec((1, 1, Dh), lambda i: (i, 0, 0)),
        ],
        out_shape=[
            jax.ShapeDtypeStruct((M, Dh), jnp.float32),
            jax.ShapeDtypeStruct((nblk, 1, Dh), jnp.float32),
            jax.ShapeDtypeStruct((nblk, 1, Dh), jnp.float32),
        ],
    )(hin, W, b.reshape(1, -1))
    mu = psum.reshape(nblk, Dh).sum(axis=0) / M
    var = psq.reshape(nblk, Dh).sum(axis=0) / M - mu * mu
    return h, mu, var


def _edge_conv_opt(x, cols, params, n):
    W0, b0, g0, be0, W1, b1, g1, be1 = params
    xi = _sc_gather(x, cols)
    xj = jnp.repeat(x, _K, axis=0)
    cat = jnp.concatenate([xi, xj - xi], axis=-1)
    h0, mu0, var0 = _mlp_layer(cat, W0, b0)
    h0 = jax.nn.relu((h0 - mu0) / jnp.sqrt(var0 + _EPS) * g0 + be0)
    h1, mu1, var1 = _mlp_layer(h0, W1, b1)
    m = jax.nn.relu((h1 - mu1) / jnp.sqrt(var1 + _EPS) * g1 + be1)
    out = jax.ops.segment_max(m, cols, num_segments=n)
    return jnp.where(jnp.isneginf(out), 0.0, out)


def _gc_post_kernel(a0_ref, a1_ref, si_ref, w_ref, b_ref, o_ref, *, relu):
    h = (a0_ref[0] + a1_ref[0]) * si_ref[...]
    out = jnp.dot(h, w_ref[...], preferred_element_type=jnp.float32) + b_ref[...]
    o_ref[...] = jnp.maximum(out, 0.0) if relu else out


def _gc_sc(x, so, si, src, dst, W, b, relu):
    n, D = x.shape
    Do = W.shape[1]
    h = x * so[:, None]
    agg2 = _sc_segsum(h, src, dst, n)
    RB = 1000
    return pl.pallas_call(
        functools.partial(_gc_post_kernel, relu=relu),
        grid=(n // RB,),
        in_specs=[
            pl.BlockSpec((1, RB, D), lambda i: (0, i, 0)),
            pl.BlockSpec((1, RB, D), lambda i: (1, i, 0)),
            pl.BlockSpec((RB, 1), lambda i: (i, 0)),
            pl.BlockSpec((D, Do), lambda i: (0, 0)),
            pl.BlockSpec((1, Do), lambda i: (0, 0)),
        ],
        out_specs=pl.BlockSpec((RB, Do), lambda i: (i, 0)),
        out_shape=jax.ShapeDtypeStruct((n, Do), jnp.float32),
    )(agg2, agg2, si[:, None], W, b.reshape(1, -1))


def _topk_kernel(y_ref, ysq_ref, xT_ref, xsq_ref, cols_ref, d2_ref, *, k, ncand):
    B = y_ref.shape[0]
    d2_ref[...] = (ysq_ref[...] + xsq_ref[...]) - 2.0 * jnp.dot(
        y_ref[...], xT_ref[...], preferred_element_type=jnp.float32)
    m_prev = jnp.full((B, 1), -jnp.inf, jnp.float32)
    j_prev = jnp.full((B, 1), -1, jnp.int32)
    outs = []
    for _ in range(k):
        d2 = d2_ref[...]
        jj = jax.lax.broadcasted_iota(jnp.int32, (B, ncand), 1)
        elig = (d2 > m_prev) | ((d2 == m_prev) & (jj > j_prev))
        m = jnp.min(jnp.where(elig, d2, jnp.inf), axis=1, keepdims=True)
        jsel = jnp.min(
            jnp.where((d2 == m) & ((m > m_prev) | (jj > j_prev)), jj, ncand),
            axis=1, keepdims=True)
        outs.append(jsel)
        m_prev, j_prev = m, jsel
    cols_ref[...] = jnp.concatenate(outs, axis=1)


def _knn_cols(x, k, B=400):
    n, d = x.shape
    dp = 16 if d <= 16 else 32
    ncand = ((n + 127) // 128) * 128
    xp = jnp.pad(x, ((0, 0), (0, dp - d)))
    xT = jnp.pad(xp.T, ((0, 0), (0, ncand - n)))
    xsq = jnp.sum(x * x, axis=1)
    xsq_row = jnp.pad(xsq[None, :], ((0, 0), (0, ncand - n)),
                      constant_values=jnp.inf)
    ysq_col = xsq[:, None]
    return pl.pallas_call(
        functools.partial(_topk_kernel, k=k, ncand=ncand),
        grid=(n // B,),
        in_specs=[
            pl.BlockSpec((B, dp), lambda i: (i, 0)),
            pl.BlockSpec((B, 1), lambda i: (i, 0)),
            pl.BlockSpec((dp, ncand), lambda i: (0, 0)),
            pl.BlockSpec((1, ncand), lambda i: (0, 0)),
        ],
        out_specs=pl.BlockSpec((B, k), lambda i: (i, 0)),
        out_shape=jax.ShapeDtypeStruct((n, k), jnp.int32),
        scratch_shapes=[pltpu.VMEM((B, ncand), jnp.float32)],
    )(xp, ysq_col, xT, xsq_row)


def _knn_flat(x, k):
    x = jax.lax.stop_gradient(x)
    mn = x.min()
    x = x - mn
    mx = x.max()
    x = x / mx
    x = jnp.concatenate([x, jnp.zeros((x.shape[0], 1), x.dtype)], axis=1)
    n = x.shape[0]
    return _knn_cols(x, k).reshape(n * k)


def kernel(features, edge_index, W1, b1, W2, b2, W3, b3, W4, b4, ec1_W0, ec1_b0, ec1_g0, ec1_be0, ec1_W1, ec1_b1, ec1_g1, ec1_be1, ec2_W0, ec2_b0, ec2_g0, ec2_be0, ec2_W1, ec2_b1, ec2_g1, ec2_be1):
    n = features.shape[0]
    src, dst = edge_index[0], edge_index[1]
    deg_out = jnp.maximum(jnp.bincount(src, length=n).astype(features.dtype), 1.0)
    deg_in = jnp.maximum(jnp.bincount(dst, length=n).astype(features.dtype), 1.0)
    so = deg_out ** -0.5
    si = deg_in ** -0.5
    h = _gc_sc(features, so, si, src, dst, W1, b1, True)
    cols = _knn_flat(h, _K)
    h = _edge_conv_opt(h, cols, (ec1_W0, ec1_b0, ec1_g0, ec1_be0, ec1_W1, ec1_b1, ec1_g1, ec1_be1), n)
    h = _gc_sc(h, so, si, src, dst, W2, b2, True)
    cols = _knn_flat(h, _K)
    h = _edge_conv_opt(h, cols, (ec2_W0, ec2_b0, ec2_g0, ec2_be0, ec2_W1, ec2_b1, ec2_g1, ec2_be1), n)
    h = _gc_sc(h, so, si, src, dst, W3, b3, True)
    h = _gc_sc(h, so, si, src, dst, W4, b4, False)
    return h


# speedup vs baseline: 1.0949x; 1.0949x over previous
import functools

import jax
import jax.numpy as jnp
from jax import lax
from jax.experimental import pallas as pl
from jax.experimental.pallas import tpu as pltpu
from jax.experimental.pallas import tpu_sc as plsc

_EPS = 1e-05
_K = 20
_NSUB = 16


def _sc_segsum(vals, src, dst, n_out):
    N, D = vals.shape
    E = src.shape[0]
    NW = 2 * _NSUB
    epw = E // NW
    nfull = epw // 128
    tail = epw - nfull * 128
    rpt = ((n_out + _NSUB * 8 - 1) // (_NSUB * 8)) * 8
    n_pad = rpt * _NSUB
    zeros = jnp.zeros((rpt, D), jnp.float32)

    scratch = [
        pltpu.VMEM((128,), jnp.int32),
        pltpu.VMEM((128,), jnp.int32),
        pltpu.VMEM((128, D), jnp.float32),
        pltpu.VMEM_SHARED((n_pad, D), jnp.float32),
        pltpu.SemaphoreType.DMA,
    ]
    if tail:
        scratch += [
            pltpu.VMEM((tail,), jnp.int32),
            pltpu.VMEM((tail,), jnp.int32),
            pltpu.VMEM((tail, D), jnp.float32),
        ]

    @functools.partial(
        pl.kernel,
        out_type=jax.ShapeDtypeStruct((2, n_pad, D), jnp.float32),
        mesh=plsc.VectorSubcoreMesh(core_axis_name="c", subcore_axis_name="s"),
        scratch_types=scratch,
        compiler_params=pltpu.CompilerParams(use_tc_tiling_on_sc=False),
    )
    def k(vals_hbm, src_hbm, dst_hbm, zeros_hbm, out_hbm,
          src_v, dst_v, rows_v, acc, sem0, *tailrefs):
        c = lax.axis_index("c")
        s = lax.axis_index("s")
        wid = s * 2 + c
        pltpu.sync_copy(zeros_hbm, acc.at[pl.ds(s * rpt, rpt)])
        plsc.subcore_barrier()
        base = wid * epw

        def body(i, carry):
            off = base + i * 128
            pltpu.sync_copy(src_hbm.at[pl.ds(off, 128)], src_v)
            pltpu.sync_copy(dst_hbm.at[pl.ds(off, 128)], dst_v)
            pltpu.async_copy(vals_hbm.at[src_v], rows_v, sem0).wait()
            pltpu.sync_copy(rows_v, acc.at[dst_v], add=True)
            return carry

        lax.fori_loop(0, nfull, body, 0)
        if tail:
            src_t, dst_t, rows_t = tailrefs
            off = base + nfull * 128
            pltpu.sync_copy(src_hbm.at[pl.ds(off, tail)], src_t)
            pltpu.sync_copy(dst_hbm.at[pl.ds(off, tail)], dst_t)
            pltpu.async_copy(vals_hbm.at[src_t], rows_t, sem0).wait()
            pltpu.sync_copy(rows_t, acc.at[dst_t], add=True)
        plsc.subcore_barrier()
        pltpu.sync_copy(acc.at[pl.ds(s * rpt, rpt)],
                        out_hbm.at[c, pl.ds(s * rpt, rpt)])

    return k(vals, src, dst, zeros)


def _sc_gather(vals, idx):
    N, D = vals.shape
    M = idx.shape[0]
    NW = 2 * _NSUB
    nch = M // 128
    tail = M - nch * 128
    cpw = (nch + NW - 1) // NW

    scratch = [
        pltpu.VMEM((128,), jnp.int32),
        pltpu.VMEM((128, D), jnp.float32),
        pltpu.SemaphoreType.DMA,
    ]
    if tail:
        scratch += [
            pltpu.VMEM((tail,), jnp.int32),
            pltpu.VMEM((tail, D), jnp.float32),
        ]

    @functools.partial(
        pl.kernel,
        out_type=jax.ShapeDtypeStruct((M, D), jnp.float32),
        mesh=plsc.VectorSubcoreMesh(core_axis_name="c", subcore_axis_name="s"),
        scratch_types=scratch,
        compiler_params=pltpu.CompilerParams(use_tc_tiling_on_sc=False),
    )
    def k(vals_hbm, idx_hbm, out_hbm, idx_v, rows_v, sem0, *tailrefs):
        c = lax.axis_index("c")
        s = lax.axis_index("s")
        wid = s * 2 + c

        def body(i, carry):
            g = wid + i * NW

            @pl.when(g < nch)
            def _():
                off = g * 128
                pltpu.sync_copy(idx_hbm.at[pl.ds(off, 128)], idx_v)
                pltpu.async_copy(vals_hbm.at[idx_v], rows_v, sem0).wait()
                pltpu.sync_copy(rows_v, out_hbm.at[pl.ds(off, 128)])

            return carry

        lax.fori_loop(0, cpw, body, 0)
        if tail:
            idx_t, rows_t = tailrefs

            @pl.when(wid == NW - 1)
            def _():
                off = nch * 128
                pltpu.sync_copy(idx_hbm.at[pl.ds(off, tail)], idx_t)
                pltpu.async_copy(vals_hbm.at[idx_t], rows_t, sem0).wait()
                pltpu.sync_copy(rows_t, out_hbm.at[pl.ds(off, tail)])

    return k(vals, idx)


def _mlp_layer_kernel(hin_ref, w_ref, b_ref, hout_ref, psum_ref, psq_ref):
    h = jnp.dot(hin_ref[...], w_ref[...], preferred_element_type=jnp.float32) + b_ref[...]
    hout_ref[...] = h
    psum_ref[0, 0, :] = jnp.sum(h, axis=0)
    psq_ref[0, 0, :] = jnp.sum(h * h, axis=0)


def _mlp_layer(hin, W, b, EB=4000):
    M, _ = hin.shape
    Dh = W.shape[1]
    Din = hin.shape[1]
    nblk = M // EB
    h, psum, psq = pl.pallas_call(
        _mlp_layer_kernel,
        grid=(nblk,),
        in_specs=[
            pl.BlockSpec((EB, Din), lambda i: (i, 0)),
            pl.BlockSpec((Din, Dh), lambda i: (0, 0)),
            pl.BlockSpec((1, Dh), lambda i: (0, 0)),
        ],
        out_specs=[
            pl.BlockSpec((EB, Dh), lambda i: (i, 0)),
            pl.BlockSpec((1, 1, Dh), lambda i: (i, 0, 0)),
            pl.BlockSpec((1, 1, Dh), lambda i: (i, 0, 0)),
        ],
        out_shape=[
            jax.ShapeDtypeStruct((M, Dh), jnp.float32),
            jax.ShapeDtypeStruct((nblk, 1, Dh), jnp.float32),
            jax.ShapeDtypeStruct((nblk, 1, Dh), jnp.float32),
        ],
    )(hin, W, b.reshape(1, -1))
    mu = psum.reshape(nblk, Dh).sum(axis=0) / M
    var = psq.reshape(nblk, Dh).sum(axis=0) / M - mu * mu
    return h, mu, var


def _edge_conv_opt(x, cols, params, n):
    W0, b0, g0, be0, W1, b1, g1, be1 = params
    xi = _sc_gather(x, cols)
    xj = jnp.repeat(x, _K, axis=0)
    cat = jnp.concatenate([xi, xj - xi], axis=-1)
    h0, mu0, var0 = _mlp_layer(cat, W0, b0)
    h0 = jax.nn.relu((h0 - mu0) / jnp.sqrt(var0 + _EPS) * g0 + be0)
    h1, mu1, var1 = _mlp_layer(h0, W1, b1)
    m = jax.nn.relu((h1 - mu1) / jnp.sqrt(var1 + _EPS) * g1 + be1)
    out = jax.ops.segment_max(m, cols, num_segments=n)
    return jnp.where(jnp.isneginf(out), 0.0, out)


def _gc_post_kernel(a0_ref, a1_ref, si_ref, w_ref, b_ref, o_ref, *, relu):
    h = (a0_ref[0] + a1_ref[0]) * si_ref[...]
    out = jnp.dot(h, w_ref[...], preferred_element_type=jnp.float32) + b_ref[...]
    o_ref[...] = jnp.maximum(out, 0.0) if relu else out


def _gc_sc(x, so, si, src, dst, W, b, relu):
    n, D = x.shape
    Do = W.shape[1]
    h = x * so[:, None]
    agg2 = _sc_segsum(h, src, dst, n)
    RB = 1000
    return pl.pallas_call(
        functools.partial(_gc_post_kernel, relu=relu),
        grid=(n // RB,),
        in_specs=[
            pl.BlockSpec((1, RB, D), lambda i: (0, i, 0)),
            pl.BlockSpec((1, RB, D), lambda i: (1, i, 0)),
            pl.BlockSpec((RB, 1), lambda i: (i, 0)),
            pl.BlockSpec((D, Do), lambda i: (0, 0)),
            pl.BlockSpec((1, Do), lambda i: (0, 0)),
        ],
        out_specs=pl.BlockSpec((RB, Do), lambda i: (i, 0)),
        out_shape=jax.ShapeDtypeStruct((n, Do), jnp.float32),
    )(agg2, agg2, si[:, None], W, b.reshape(1, -1))


def _topk_kernel(y_ref, ysq_ref, xT_ref, xsq_ref, cols_ref, d2_ref, *, k, ncand):
    B = y_ref.shape[0]
    d2_ref[...] = (ysq_ref[...] + xsq_ref[...]) - 2.0 * jnp.dot(
        y_ref[...], xT_ref[...], preferred_element_type=jnp.float32)
    m_prev = jnp.full((B, 1), -jnp.inf, jnp.float32)
    j_prev = jnp.full((B, 1), -1, jnp.int32)
    outs = []
    for _ in range(k):
        d2 = d2_ref[...]
        jj = jax.lax.broadcasted_iota(jnp.int32, (B, ncand), 1)
        elig = (d2 > m_prev) | ((d2 == m_prev) & (jj > j_prev))
        m = jnp.min(jnp.where(elig, d2, jnp.inf), axis=1, keepdims=True)
        jsel = jnp.min(jnp.where(elig & (d2 == m), jj, ncand), axis=1, keepdims=True)
        outs.append(jsel)
        m_prev, j_prev = m, jsel
    cols_ref[...] = jnp.concatenate(outs, axis=1)


def _knn_cols(x, k, B=400):
    n, d = x.shape
    dp = 16 if d <= 16 else 32
    ncand = ((n + 127) // 128) * 128
    xp = jnp.pad(x, ((0, 0), (0, dp - d)))
    xT = jnp.pad(xp.T, ((0, 0), (0, ncand - n)))
    xsq = jnp.sum(x * x, axis=1)
    xsq_row = jnp.pad(xsq[None, :], ((0, 0), (0, ncand - n)),
                      constant_values=jnp.inf)
    ysq_col = xsq[:, None]
    return pl.pallas_call(
        functools.partial(_topk_kernel, k=k, ncand=ncand),
        grid=(n // B,),
        in_specs=[
            pl.BlockSpec((B, dp), lambda i: (i, 0)),
            pl.BlockSpec((B, 1), lambda i: (i, 0)),
            pl.BlockSpec((dp, ncand), lambda i: (0, 0)),
            pl.BlockSpec((1, ncand), lambda i: (0, 0)),
        ],
        out_specs=pl.BlockSpec((B, k), lambda i: (i, 0)),
        out_shape=jax.ShapeDtypeStruct((n, k), jnp.int32),
        scratch_shapes=[pltpu.VMEM((B, ncand), jnp.float32)],
    )(xp, ysq_col, xT, xsq_row)


def _knn_flat(x, k):
    x = jax.lax.stop_gradient(x)
    mn = x.min()
    x = x - mn
    mx = x.max()
    x = x / mx
    x = jnp.concatenate([x, jnp.zeros((x.shape[0], 1), x.dtype)], axis=1)
    n = x.shape[0]
    return _knn_cols(x, k).reshape(n * k)


def kernel(features, edge_index, W1, b1, W2, b2, W3, b3, W4, b4, ec1_W0, ec1_b0, ec1_g0, ec1_be0, ec1_W1, ec1_b1, ec1_g1, ec1_be1, ec2_W0, ec2_b0, ec2_g0, ec2_be0, ec2_W1, ec2_b1, ec2_g1, ec2_be1):
    n = features.shape[0]
    src, dst = edge_index[0], edge_index[1]
    deg_out = jnp.maximum(jnp.bincount(src, length=n).astype(features.dtype), 1.0)
    deg_in = jnp.maximum(jnp.bincount(dst, length=n).astype(features.dtype), 1.0)
    so = deg_out ** -0.5
    si = deg_in ** -0.5
    h = _gc_sc(features, so, si, src, dst, W1, b1, True)
    cols = _knn_flat(h, _K)
    h = _edge_conv_opt(h, cols, (ec1_W0, ec1_b0, ec1_g0, ec1_be0, ec1_W1, ec1_b1, ec1_g1, ec1_be1), n)
    h = _gc_sc(h, so, si, src, dst, W2, b2, True)
    cols = _knn_flat(h, _K)
    h = _edge_conv_opt(h, cols, (ec2_W0, ec2_b0, ec2_g0, ec2_be0, ec2_W1, ec2_b1, ec2_g1, ec2_be1), n)
    h = _gc_sc(h, so, si, src, dst, W3, b3, True)
    h = _gc_sc(h, so, si, src, dst, W4, b4, False)
    return h
